# baseline (speedup 1.0000x reference)
import functools

import jax
import jax.numpy as jnp
from jax import lax
from jax.experimental import pallas as pl
from jax.experimental.pallas import tpu as pltpu
from jax.experimental.pallas import tpu_sc as plsc

VOCAB = 10000
D = 64
BATCH = 4096
SEQ = 200

NC = 2
NS = 16
NW = NC * NS

TOKENS = BATCH * SEQ
PER_W = TOKENS // NW
CK = 128
NCHUNK = PER_W // CK
NSLOT = 8
NBUFG = 4

_mesh = plsc.VectorSubcoreMesh(
    core_axis_name="c", subcore_axis_name="s", num_cores=NC, num_subcores=NS
)


@functools.partial(
    pl.kernel,
    out_type=jax.ShapeDtypeStruct((TOKENS, D), jnp.float32),
    mesh=_mesh,
    scratch_types=[
        pltpu.VMEM((NCHUNK, CK), jnp.int32),
        pltpu.VMEM((NSLOT, CK, D), jnp.float32),
        pltpu.SemaphoreType.DMA,
        pltpu.SemaphoreType.DMA,
    ],
    compiler_params=pltpu.CompilerParams(use_tc_tiling_on_sc=False),
)
def _embed_lookup(idx_hbm, table_hbm, out_hbm, idx_v, bufs, gsem, osem):
    wid = lax.axis_index("s") * NC + lax.axis_index("c")
    crow = wid * NCHUNK
    base = wid * PER_W

    pltpu.sync_copy(idx_hbm.at[pl.ds(crow, NCHUNK)], idx_v)

    for b in range(NBUFG):
        pltpu.async_copy(table_hbm.at[idx_v.at[b]], bufs.at[b], gsem)

    @pl.loop(0, NCHUNK, step=NSLOT)
    def _(g):
        for b in range(NSLOT):
            j = g + b
            pltpu.make_async_copy(
                table_hbm.at[idx_v.at[j]], bufs.at[b], gsem
            ).wait()
            pltpu.async_copy(
                bufs.at[b], out_hbm.at[pl.ds(base + j * CK, CK)], osem
            )

            @pl.when(j >= NBUFG)
            def _():
                pltpu.make_async_copy(
                    bufs.at[b], out_hbm.at[pl.ds(base, CK)], osem
                ).wait()

            @pl.when(j + NBUFG < NCHUNK)
            def _():
                pltpu.async_copy(
                    table_hbm.at[idx_v.at[j + NBUFG]],
                    bufs.at[(b + NBUFG) % NSLOT],
                    gsem,
                )

    for b in range(NBUFG):
        pltpu.make_async_copy(
            bufs.at[b], out_hbm.at[pl.ds(base, CK)], osem
        ).wait()


def kernel(token_ids, embedding_table):
    idx = jnp.asarray(token_ids, jnp.int32).reshape(TOKENS // CK, CK)
    out = _embed_lookup(idx, embedding_table)
    return out.reshape(BATCH, SEQ, D)


# speedup vs baseline: 1.0034x; 1.0034x over previous
import functools

import jax
import jax.numpy as jnp
from jax import lax
from jax.experimental import pallas as pl
from jax.experimental.pallas import tpu as pltpu
from jax.experimental.pallas import tpu_sc as plsc

VOCAB = 10000
D = 64
BATCH = 4096
SEQ = 200

NC = 2
NS = 16
NW = NC * NS

TOKENS = BATCH * SEQ
PER_W = TOKENS // NW
CK = 128
NCHUNK = PER_W // CK
NSLOT = 10
NBUFG = 5

_mesh = plsc.VectorSubcoreMesh(
    core_axis_name="c", subcore_axis_name="s", num_cores=NC, num_subcores=NS
)


@functools.partial(
    pl.kernel,
    out_type=jax.ShapeDtypeStruct((TOKENS, D), jnp.float32),
    mesh=_mesh,
    scratch_types=[
        pltpu.VMEM((NCHUNK, CK), jnp.int32),
        pltpu.VMEM((NSLOT, CK, D), jnp.float32),
        pltpu.SemaphoreType.DMA,
        pltpu.SemaphoreType.DMA,
    ],
    compiler_params=pltpu.CompilerParams(use_tc_tiling_on_sc=False),
)
def _embed_lookup(idx_hbm, table_hbm, out_hbm, idx_v, bufs, gsem, osem):
    sid = lax.axis_index("s")
    wid = sid * NC + lax.axis_index("c")
    crow = wid * NCHUNK
    base = wid * PER_W

    pltpu.sync_copy(idx_hbm.at[pl.ds(crow, NCHUNK)], idx_v)

    for b in range(NBUFG):
        pltpu.async_copy(table_hbm.at[idx_v.at[b]], bufs.at[b], gsem)

    @pl.loop(0, NCHUNK, step=NSLOT)
    def _(g):
        for b in range(NSLOT):
            j = g + b
            pltpu.make_async_copy(
                table_hbm.at[idx_v.at[j]], bufs.at[b], gsem
            ).wait()
            pltpu.async_copy(
                bufs.at[b], out_hbm.at[pl.ds(base + j * CK, CK)], osem
            )

            @pl.when(j >= NBUFG)
            def _():
                pltpu.make_async_copy(
                    bufs.at[b], out_hbm.at[pl.ds(base, CK)], osem
                ).wait()

            @pl.when(j + NBUFG < NCHUNK)
            def _():
                pltpu.async_copy(
                    table_hbm.at[idx_v.at[j + NBUFG]],
                    bufs.at[(b + NBUFG) % NSLOT],
                    gsem,
                )

    for b in range(NBUFG):
        pltpu.make_async_copy(
            bufs.at[b], out_hbm.at[pl.ds(base, CK)], osem
        ).wait()


def kernel(token_ids, embedding_table):
    idx = jnp.asarray(token_ids, jnp.int32).reshape(TOKENS // CK, CK)
    out = _embed_lookup(idx, embedding_table)
    return out.reshape(BATCH, SEQ, D)
